# baseline (speedup 1.0000x reference)
import functools

import jax
import jax.numpy as jnp
from jax import lax
from jax.experimental import pallas as pl
from jax.experimental.pallas import tpu as pltpu
from jax.experimental.pallas import tpu_sc as plsc

_NC = 2
_NS = 16
_NW = _NC * _NS
_CH = 128
_NBUF = 2


def _sc_add_pos(x_hbm, pos_hbm, out_hbm, buf, idx_v, pos_sh,
                sem_in, sem_out):
    rows = x_hbm.shape[0]
    rows_per_w = rows // _NW
    n_chunks = rows_per_w // _CH

    cid = lax.axis_index("c")
    sid = lax.axis_index("s")
    wid = sid * _NC + cid
    w_base = wid * rows_per_w

    @pl.when(sid == 0)
    def _():
        pltpu.sync_copy(pos_hbm.at[pl.ds(0, _CH)], pos_sh)

    for i in range(_CH // 16):
        idx_v[pl.ds(i * 16, 16)] = lax.iota(jnp.int32, 16) + i * 16

    plsc.subcore_barrier()

    def start_in(c, b):
        pltpu.async_copy(
            x_hbm.at[pl.ds(w_base + c * _CH, _CH)], buf.at[b], sem_in.at[b])

    def wait_in(c, b):
        pltpu.make_async_copy(
            x_hbm.at[pl.ds(w_base + c * _CH, _CH)], buf.at[b],
            sem_in.at[b]).wait()

    def start_out(c, b):
        pltpu.async_copy(
            buf.at[b], out_hbm.at[pl.ds(w_base + c * _CH, _CH)],
            sem_out.at[b])

    def wait_out(c, b):
        pltpu.make_async_copy(
            buf.at[b], out_hbm.at[pl.ds(w_base + c * _CH, _CH)],
            sem_out.at[b]).wait()

    start_in(0, 0)

    def body(c, _):
        b = lax.rem(c, _NBUF)
        nb = lax.rem(c + 1, _NBUF)

        @pl.when(c >= 1)
        def _():
            wait_out(c - 1, nb)

        @pl.when(c + 1 < n_chunks)
        def _():
            start_in(c + 1, nb)

        wait_in(c, b)
        pltpu.sync_copy(pos_sh.at[idx_v], buf.at[b], add=True)
        start_out(c, b)
        return 0

    lax.fori_loop(0, n_chunks, body, 0)
    wait_out(n_chunks - 1, lax.rem(n_chunks - 1, _NBUF))


def kernel(x, pos_emb):
    B, L, D = x.shape
    x2 = x.reshape(B * L, D)
    run = functools.partial(
        pl.kernel,
        out_type=jax.ShapeDtypeStruct((B * L, D), x.dtype),
        mesh=plsc.VectorSubcoreMesh(core_axis_name="c", subcore_axis_name="s"),
        scratch_types=[
            pltpu.VMEM((_NBUF, _CH, D), jnp.float32),
            pltpu.VMEM((_CH,), jnp.int32),
            pltpu.VMEM_SHARED((_CH, D), jnp.float32),
            pltpu.SemaphoreType.DMA((_NBUF,)),
            pltpu.SemaphoreType.DMA((_NBUF,)),
        ],
    )(_sc_add_pos)
    out = run(x2, pos_emb)
    return out.reshape(B, L, D)


# speedup vs baseline: 1.0509x; 1.0509x over previous
import functools

import jax
import jax.numpy as jnp
from jax import lax
from jax.experimental import pallas as pl
from jax.experimental.pallas import tpu as pltpu
from jax.experimental.pallas import tpu_sc as plsc

_NC = 2
_NS = 16
_NW = _NC * _NS
_CH = 128
_NBUF = 3


def _sc_add_pos(x_hbm, pos_hbm, out_hbm, buf, pos_v, sem_in, sem_out):
    rows = x_hbm.shape[0]
    rows_per_w = rows // _NW
    n_chunks = rows_per_w // _CH

    cid = lax.axis_index("c")
    sid = lax.axis_index("s")
    wid = sid * _NC + cid
    w_base = wid * rows_per_w

    pltpu.sync_copy(pos_hbm.at[pl.ds(0, _CH)], pos_v)

    def start_in(c, b):
        pltpu.async_copy(
            x_hbm.at[pl.ds(w_base + c * _CH, _CH)], buf.at[b], sem_in.at[b])

    def wait_in(c, b):
        pltpu.make_async_copy(
            x_hbm.at[pl.ds(w_base + c * _CH, _CH)], buf.at[b],
            sem_in.at[b]).wait()

    def start_out(c, b):
        pltpu.async_copy(
            buf.at[b], out_hbm.at[pl.ds(w_base + c * _CH, _CH)],
            sem_out.at[b])

    def wait_out(c, b):
        pltpu.make_async_copy(
            buf.at[b], out_hbm.at[pl.ds(w_base + c * _CH, _CH)],
            sem_out.at[b]).wait()

    start_in(0, 0)
    start_in(1, 1)

    def body(c, _):
        b = lax.rem(c, _NBUF)
        wait_in(c, b)

        @plsc.parallel_loop(0, _CH)
        def _(r):
            for j in range(_CH // 16):
                sl = pl.ds(j * 16, 16)
                plsc.addupdate(buf.at[b, r, sl], pos_v[r, sl])

        bp = lax.rem(c + 2, _NBUF)

        @pl.when(c >= 1)
        def _():
            wait_out(c - 1, bp)

        @pl.when(c + 2 < n_chunks)
        def _():
            start_in(c + 2, bp)

        start_out(c, b)
        return 0

    lax.fori_loop(0, n_chunks, body, 0)
    wait_out(n_chunks - 1, lax.rem(n_chunks - 1, _NBUF))


def kernel(x, pos_emb):
    B, L, D = x.shape
    x2 = x.reshape(B * L, D)
    run = functools.partial(
        pl.kernel,
        out_type=jax.ShapeDtypeStruct((B * L, D), x.dtype),
        mesh=plsc.VectorSubcoreMesh(core_axis_name="c", subcore_axis_name="s"),
        scratch_types=[
            pltpu.VMEM((_NBUF, _CH, D), jnp.float32),
            pltpu.VMEM((_CH, D), jnp.float32),
            pltpu.SemaphoreType.DMA((_NBUF,)),
            pltpu.SemaphoreType.DMA((_NBUF,)),
        ],
    )(_sc_add_pos)
    out = run(x2, pos_emb)
    return out.reshape(B, L, D)


# speedup vs baseline: 1.1129x; 1.0590x over previous
import functools

import jax
import jax.numpy as jnp
from jax import lax
from jax.experimental import pallas as pl
from jax.experimental.pallas import tpu as pltpu
from jax.experimental.pallas import tpu_sc as plsc

_NC = 2
_NS = 16
_NW = _NC * _NS
_L = 128
_D = 128
_CH = 256
_NBUF = 3


def _sc_add_pos(x_hbm, pos_hbm, out_hbm, buf, pos_v, sem_in, sem_out):
    rows = x_hbm.shape[0]
    rows_per_w = rows // _NW
    n_chunks = rows_per_w // _CH

    cid = lax.axis_index("c")
    sid = lax.axis_index("s")
    wid = sid * _NC + cid
    w_base = wid * rows_per_w

    pltpu.sync_copy(pos_hbm.at[pl.ds(0, _L)], pos_v)

    def start_in(c, b):
        pltpu.async_copy(
            x_hbm.at[pl.ds(w_base + c * _CH, _CH)], buf.at[b], sem_in.at[b])

    def wait_in(c, b):
        pltpu.make_async_copy(
            x_hbm.at[pl.ds(w_base + c * _CH, _CH)], buf.at[b],
            sem_in.at[b]).wait()

    def start_out(c, b):
        pltpu.async_copy(
            buf.at[b], out_hbm.at[pl.ds(w_base + c * _CH, _CH)],
            sem_out.at[b])

    def wait_out(c, b):
        pltpu.make_async_copy(
            buf.at[b], out_hbm.at[pl.ds(w_base + c * _CH, _CH)],
            sem_out.at[b]).wait()

    start_in(0, 0)
    start_in(1, 1)

    def body(c, _):
        b = lax.rem(c, _NBUF)
        wait_in(c, b)

        @plsc.parallel_loop(0, _CH)
        def _(r):
            for j in range(_D // 16):
                sl = pl.ds(j * 16, 16)
                plsc.addupdate(buf.at[b, r, sl], pos_v[lax.rem(r, _L), sl])

        bp = lax.rem(c + 2, _NBUF)

        @pl.when(c >= 1)
        def _():
            wait_out(c - 1, bp)

        @pl.when(c + 2 < n_chunks)
        def _():
            start_in(c + 2, bp)

        start_out(c, b)
        return 0

    lax.fori_loop(0, n_chunks, body, 0)
    wait_out(n_chunks - 1, lax.rem(n_chunks - 1, _NBUF))


def kernel(x, pos_emb):
    B, L, D = x.shape
    x2 = x.reshape(B * L, D)
    run = functools.partial(
        pl.kernel,
        out_type=jax.ShapeDtypeStruct((B * L, D), x.dtype),
        mesh=plsc.VectorSubcoreMesh(core_axis_name="c", subcore_axis_name="s"),
        scratch_types=[
            pltpu.VMEM((_NBUF, _CH, D), jnp.float32),
            pltpu.VMEM((_L, D), jnp.float32),
            pltpu.SemaphoreType.DMA((_NBUF,)),
            pltpu.SemaphoreType.DMA((_NBUF,)),
        ],
    )(_sc_add_pos)
    out = run(x2, pos_emb)
    return out.reshape(B, L, D)
